# baseline (speedup 1.0000x reference)
import functools

import jax
import jax.numpy as jnp
from jax import lax
from jax.experimental import pallas as pl
from jax.experimental.pallas import tpu as pltpu

B = 64
NPATCH = 256
DIM = 384
HID = 1536
KEEP1 = 307
KEEP2 = 1228
K2PAD = 1280


def _masks(s_c, tri_ref, d, keep, chunk):
    f32 = jnp.float32
    s_r = jnp.transpose(s_c, (1, 0))
    nch = d // chunk
    rank_r = jnp.zeros((1, d), f32)
    for ic in range(nch):
        s_ci = lax.slice(s_c, (ic * chunk, 0), ((ic + 1) * chunk, 1))
        il = lax.broadcasted_iota(jnp.int32, (chunk, d), 1)
        isub = lax.broadcasted_iota(jnp.int32, (chunk, d), 0) + ic * chunk
        cmp = (s_ci > s_r) | ((s_ci == s_r) & (isub < il))
        rank_r = rank_r + jnp.sum(cmp.astype(f32), axis=0, keepdims=True)
    kept_r = rank_r < keep
    kept_bf = jnp.where(kept_r, 1.0, 0.0).astype(jnp.bfloat16)
    tri = tri_ref[0:d, 0:d]
    pos_r = lax.dot_general(kept_bf, tri, (((1,), (0,)), ((), ())),
                            preferred_element_type=f32)
    pos_r = pos_r.astype(jnp.int32)
    return kept_r, pos_r


PAIR = 2


def _body(x_ref, ws1c_ref, w1p_ref, b1c_ref, ws2c_ref, w2tp_ref, b2r_ref,
          tri_ref, out_ref, hc_ref):
    for i in range(PAIR):
        _one_sample(i, x_ref, ws1c_ref, w1p_ref, b1c_ref, ws2c_ref,
                    w2tp_ref, b2r_ref, tri_ref, out_ref, hc_ref)


def _one_sample(i, x_ref, ws1c_ref, w1p_ref, b1c_ref, ws2c_ref, w2tp_ref,
                b2r_ref, tri_ref, out_ref, hc_ref):
    f32 = jnp.float32
    xb = x_ref[i]

    a1c = lax.dot_general(xb, ws1c_ref[...], (((0,), (0,)), ((), ())),
                          preferred_element_type=f32)
    kept1_r, pos1_r = _masks(a1c, tri_ref, DIM, KEEP1, DIM)

    isub = lax.broadcasted_iota(jnp.int32, (DIM, DIM), 0)
    p1t = jnp.where(kept1_r & (pos1_r == isub), 1.0, 0.0).astype(f32)
    xc = lax.dot_general(xb, p1t, (((1,), (1,)), ((), ())),
                         preferred_element_type=f32)

    hT = lax.dot_general(w1p_ref[...], xc, (((1,), (1,)), ((), ())),
                         preferred_element_type=f32)
    hT = jnp.maximum(hT + b1c_ref[...], 0.0)

    a2c = lax.dot_general(hT, ws2c_ref[...], (((1,), (0,)), ((), ())),
                          preferred_element_type=f32)
    kept2_r, pos2_r = _masks(a2c, tri_ref, HID, KEEP2, 512)

    hc_ref[i] = jnp.zeros((HID, NPATCH), f32)
    for sb in range(HID // 256):
        w0 = max(0, sb * 256 - 320)
        k2b = lax.slice(kept2_r, (0, sb * 256), (1, (sb + 1) * 256))
        p2b = lax.slice(pos2_r, (0, sb * 256), (1, (sb + 1) * 256))
        isub = lax.broadcasted_iota(jnp.int32, (576, 256), 0) + w0
        p2t = jnp.where(k2b & (p2b == isub), 1.0, 0.0).astype(f32)
        hblk = lax.slice(hT, (sb * 256, 0), ((sb + 1) * 256, NPATCH))
        contrib = lax.dot_general(p2t, hblk, (((1,), (0,)), ((), ())),
                                  preferred_element_type=f32)
        hc_ref[i, w0:w0 + 576, :] = hc_ref[i, w0:w0 + 576, :] + contrib
    hc = hc_ref[i, 0:K2PAD, :]
    out = lax.dot_general(hc, w2tp_ref[...], (((0,), (0,)), ((), ())),
                          preferred_element_type=f32)
    out_ref[i] = out + b2r_ref[...]


@functools.partial(jax.jit, static_argnames=("interpret",))
def kernel(x, Ws1, bs1, W1, b1, Ws2, bs2, W2, b2, interpret=False):
    f32 = jnp.float32
    ws1c = Ws1.reshape(NPATCH, 1).astype(f32)
    w1p = jnp.pad(W1, ((0, 0), (0, DIM - KEEP1))).astype(f32)
    b1c = b1.reshape(HID, 1).astype(f32)
    ws2c = Ws2.reshape(NPATCH, 1).astype(f32)
    w2tp = jnp.pad(W2.T, ((0, K2PAD - KEEP2), (0, 0))).astype(f32)
    b2r = b2.reshape(1, DIM).astype(f32)
    tri = jnp.triu(jnp.ones((HID, HID), jnp.bfloat16), 1)

    out = pl.pallas_call(
        _body,
        grid=(B // PAIR,),
        in_specs=[
            pl.BlockSpec((PAIR, NPATCH, DIM), lambda b: (b, 0, 0)),
            pl.BlockSpec((NPATCH, 1), lambda b: (0, 0)),
            pl.BlockSpec((HID, DIM), lambda b: (0, 0)),
            pl.BlockSpec((HID, 1), lambda b: (0, 0)),
            pl.BlockSpec((NPATCH, 1), lambda b: (0, 0)),
            pl.BlockSpec((K2PAD, DIM), lambda b: (0, 0)),
            pl.BlockSpec((1, DIM), lambda b: (0, 0)),
            pl.BlockSpec((HID, HID), lambda b: (0, 0)),
        ],
        out_specs=pl.BlockSpec((PAIR, NPATCH, DIM), lambda b: (b, 0, 0)),
        out_shape=jax.ShapeDtypeStruct((B, NPATCH, DIM), f32),
        scratch_shapes=[pltpu.VMEM((PAIR, HID, NPATCH), f32)],
        interpret=interpret,
    )(x, ws1c, w1p, b1c, ws2c, w2tp, b2r, tri)
    return out
